# Initial kernel scaffold; baseline (speedup 1.0000x reference)
import jax
import jax.numpy as jnp
from jax.experimental import pallas as pl


def kernel(z, edge_index, edge_weight, batch, params):
    raise NotImplementedError("write your pallas kernel here")


# speedup vs baseline: 2.2334x; 2.2334x over previous
import functools

import jax
import jax.numpy as jnp
from jax import lax
from jax.experimental import pallas as pl
from jax.experimental.pallas import tpu as pltpu
from jax.experimental.pallas import tpu_sc as plsc

_H = 128
_NSEG = 64
_DCUT = 5.0
_CW = 128
_NW = 32
_EB = 2560


def _elu(v):
    return jnp.where(v > 0, v, jnp.exp(jnp.minimum(v, 0.0)) - 1.0)


def _embed_body(z_ref, emb_ref, ws_ref, wt_ref, x_ref, xs_ref, xt_ref):
    z = z_ref[...]
    ids = lax.broadcasted_iota(jnp.int32, (1, emb_ref.shape[0]), 1)
    oh = (z == ids).astype(jnp.float32)
    x = jnp.dot(oh, emb_ref[...], preferred_element_type=jnp.float32)
    x_ref[...] = x
    xs_ref[...] = jnp.dot(x, ws_ref[...], preferred_element_type=jnp.float32)
    xt_ref[...] = jnp.dot(x, wt_ref[...], preferred_element_type=jnp.float32)


def _sc_gather_body(xs_hbm, xt_hbm, src_hbm, dst_hbm, gs_hbm, gt_hbm,
                    idx_s, idx_t, rows_s, rows_t, sem_s, sem_t):
    wid = lax.axis_index("s") * 2 + lax.axis_index("c")
    nchunks = src_hbm.shape[0] // _CW
    iters = (nchunks + _NW - 1) // _NW

    def body(j, carry):
        c = j * _NW + wid

        @pl.when(c < nchunks)
        def _():
            pltpu.sync_copy(src_hbm.at[pl.ds(c * _CW, _CW)], idx_s)
            pltpu.sync_copy(dst_hbm.at[pl.ds(c * _CW, _CW)], idx_t)
            cp_s = pltpu.async_copy(xs_hbm.at[idx_s], rows_s, sem_s)
            cp_t = pltpu.async_copy(xt_hbm.at[idx_t], rows_t, sem_t)
            cp_s.wait()
            cp_t.wait()
            pltpu.sync_copy(rows_s, gs_hbm.at[pl.ds(c * _CW, _CW)])
            pltpu.sync_copy(rows_t, gt_hbm.at[pl.ds(c * _CW, _CW)])

        return carry

    lax.fori_loop(0, iters, body, 0)


def _edge_body(gs_ref, gt_ref, d_ref, wd_ref, w1_ref, w2_ref,
               b1_ref, b2_ref, means_ref, betas_ref, out_ref):
    d = d_ref[...]
    dcl = jnp.minimum(d, _DCUT)
    cutoff = 0.5 * (jnp.cos(dcl * (jnp.pi / _DCUT)) + 1.0)
    rbf = cutoff * jnp.exp(-betas_ref[...] * (jnp.exp(-d) - means_ref[...]) ** 2)
    dist_emb = jnp.tanh(jnp.dot(rbf, wd_ref[...], preferred_element_type=jnp.float32))
    msg = dist_emb * (gs_ref[...] + gt_ref[...])
    h = _elu(jnp.dot(msg, w1_ref[...], preferred_element_type=jnp.float32) + b1_ref[...])
    h = _elu(jnp.dot(h, w2_ref[...], preferred_element_type=jnp.float32) + b2_ref[...])
    out_ref[...] = h


def _sc_scatter_body(h_hbm, dst_hbm, zeros_hbm, out0_hbm, out1_hbm,
                     idxr, rows, acc):
    cid = lax.axis_index("c")
    sid = lax.axis_index("s")
    npad = zeros_hbm.shape[0]
    stripe = npad // 16
    sl = pl.ds(sid * stripe, stripe)
    pltpu.sync_copy(zeros_hbm.at[sl], acc.at[sl])
    plsc.subcore_barrier()

    nchunks = dst_hbm.shape[0] // _CW
    half = nchunks // 2
    iters = (half + 15) // 16

    def body(j, carry):
        k = j * 16 + sid

        @pl.when(k < half)
        def _():
            c = cid * half + k
            pltpu.sync_copy(h_hbm.at[pl.ds(c * _CW, _CW)], rows)
            pltpu.sync_copy(dst_hbm.at[pl.ds(c * _CW, _CW)], idxr)
            pltpu.sync_copy(rows, acc.at[idxr], add=True)

        return carry

    lax.fori_loop(0, iters, body, 0)
    plsc.subcore_barrier()

    @pl.when(cid == 0)
    def _():
        pltpu.sync_copy(acc.at[sl], out0_hbm.at[sl])

    @pl.when(cid == 1)
    def _():
        pltpu.sync_copy(acc.at[sl], out1_hbm.at[sl])


def _node_body(with_proj, x_ref, p0_ref, p1_ref, bc_ref, br_ref,
               wrc_ref, wca_ref, bcomb_ref, gnw_ref, gnb_ref, gnms_ref,
               *rest):
    if with_proj:
        ws_ref, wt_ref, xo_ref, xs_ref, xt_ref = rest
    else:
        (xo_ref,) = rest
    x = x_ref[...]
    n = x.shape[0]
    aggr = p0_ref[...][:n] + p1_ref[...][:n]
    pre = _elu(jnp.dot(x, wrc_ref[...], preferred_element_type=jnp.float32)
               + jnp.dot(aggr, wca_ref[...], preferred_element_type=jnp.float32)
               + bcomb_ref[...])
    seg_row = lax.broadcasted_iota(jnp.int32, (1, _NSEG), 1)
    seg_col = lax.broadcasted_iota(jnp.int32, (_NSEG, 1), 0)
    oh = (bc_ref[...] == seg_row).astype(jnp.float32)
    oht = (seg_col == br_ref[...]).astype(jnp.float32)
    cnt = jnp.maximum(jnp.sum(oht, axis=1, keepdims=True), 1.0)
    mean = jnp.dot(oht, pre, preferred_element_type=jnp.float32) / cnt
    mb = jnp.dot(oh, mean * gnms_ref[...], preferred_element_type=jnp.float32)
    xc = pre - mb
    var = jnp.dot(oht, xc * xc, preferred_element_type=jnp.float32) / cnt
    inv = lax.rsqrt(var + 1e-5)
    invb = jnp.dot(oh, inv, preferred_element_type=jnp.float32)
    xn = x + gnw_ref[...] * xc * invb + gnb_ref[...]
    xo_ref[...] = xn
    if with_proj:
        xs_ref[...] = jnp.dot(xn, ws_ref[...], preferred_element_type=jnp.float32)
        xt_ref[...] = jnp.dot(xn, wt_ref[...], preferred_element_type=jnp.float32)


def _sc_mesh():
    return plsc.VectorSubcoreMesh(core_axis_name="c", subcore_axis_name="s")


def _gather_call(n, e):
    return pl.kernel(
        _sc_gather_body,
        out_type=[jax.ShapeDtypeStruct((e, _H), jnp.float32)] * 2,
        mesh=_sc_mesh(),
        scratch_types=[
            pltpu.VMEM((_CW,), jnp.int32),
            pltpu.VMEM((_CW,), jnp.int32),
            pltpu.VMEM((_CW, _H), jnp.float32),
            pltpu.VMEM((_CW, _H), jnp.float32),
            pltpu.SemaphoreType.DMA,
            pltpu.SemaphoreType.DMA,
        ],
    )


def _scatter_call(npad, e):
    return pl.kernel(
        _sc_scatter_body,
        out_type=[jax.ShapeDtypeStruct((npad, _H), jnp.float32)] * 2,
        mesh=_sc_mesh(),
        scratch_types=[
            pltpu.VMEM((_CW,), jnp.int32),
            pltpu.VMEM((_CW, _H), jnp.float32),
            pltpu.VMEM_SHARED((npad, _H), jnp.float32),
        ],
    )


def _edge_call(e):
    grid = e // _EB
    wspec = pl.BlockSpec((_H, _H), lambda i: (0, 0))
    vspec = pl.BlockSpec((1, _H), lambda i: (0, 0))
    espec = pl.BlockSpec((_EB, _H), lambda i: (i, 0))
    return pl.pallas_call(
        _edge_body,
        grid=(grid,),
        in_specs=[espec, espec, pl.BlockSpec((_EB, 1), lambda i: (i, 0)),
                  wspec, wspec, wspec, vspec, vspec, vspec, vspec],
        out_specs=espec,
        out_shape=jax.ShapeDtypeStruct((e, _H), jnp.float32),
    )


def kernel(z, edge_index, edge_weight, batch, params):
    n = z.shape[0]
    e = edge_index.shape[1]
    npad = ((n + 127) // 128) * 128
    src = edge_index[0].astype(jnp.int32)
    dst = edge_index[1].astype(jnp.int32)
    z2 = z.astype(jnp.int32).reshape(n, 1)
    bcol = batch.astype(jnp.int32).reshape(n, 1)
    brow = batch.astype(jnp.int32).reshape(1, n)
    zeros_nh = jnp.zeros((npad, _H), jnp.float32)

    start = jnp.exp(jnp.asarray(-_DCUT, jnp.float32))
    means = jnp.linspace(start, 1.0, _H).reshape(1, _H).astype(jnp.float32)
    betas = jnp.full((1, _H), (2.0 / _H * (1.0 - start)) ** (-2), jnp.float32)

    def prep(p):
        return dict(
            wd=p["Wdist"].T,
            we_s=p["Wedge"][:, :_H].T,
            we_t=p["Wedge"][:, _H:].T,
            w1=p["W1"].T,
            w2=p["W2"].T,
            b1=p["b1"].reshape(1, _H),
            b2=p["b2"].reshape(1, _H),
            wrc=p["Wres"].T + p["Wcomb"][:, :_H].T,
            wca=p["Wcomb"][:, _H:].T,
            bcomb=p["bcomb"].reshape(1, _H),
            gnw=p["gn_w"].reshape(1, _H),
            gnb=p["gn_b"].reshape(1, _H),
            gnms=p["gn_ms"].reshape(1, _H),
        )

    lp = [prep(p) for p in params["layers"]]

    nh = jax.ShapeDtypeStruct((n, _H), jnp.float32)
    x, xs, xt = pl.pallas_call(
        _embed_body, out_shape=(nh, nh, nh),
    )(z2, params["emb"], lp[0]["we_s"], lp[0]["we_t"])

    gather = _gather_call(n, e)
    scatter = _scatter_call(npad, e)
    edge = _edge_call(e)

    for l in range(2):
        p = lp[l]
        gs, gt = gather(xs, xt, src, dst)
        h = edge(gs, gt, edge_weight, p["wd"], p["w1"], p["w2"],
                 p["b1"], p["b2"], means, betas)
        p0, p1 = scatter(h, dst, zeros_nh)
        if l == 0:
            body = functools.partial(_node_body, True)
            x, xs, xt = pl.pallas_call(body, out_shape=(nh, nh, nh))(
                x, p0, p1, bcol, brow, p["wrc"], p["wca"], p["bcomb"],
                p["gnw"], p["gnb"], p["gnms"], lp[1]["we_s"], lp[1]["we_t"])
        else:
            body = functools.partial(_node_body, False)
            x = pl.pallas_call(body, out_shape=nh)(
                x, p0, p1, bcol, brow, p["wrc"], p["wca"], p["bcomb"],
                p["gnw"], p["gnb"], p["gnms"])
    return x


# speedup vs baseline: 2.6018x; 1.1650x over previous
import functools

import jax
import jax.numpy as jnp
from jax import lax
from jax.experimental import pallas as pl
from jax.experimental.pallas import tpu as pltpu
from jax.experimental.pallas import tpu_sc as plsc

_H = 128
_NSEG = 64
_DCUT = 5.0
_CW = 128
_NW = 32
_EB = 2560


def _elu(v):
    return jnp.where(v > 0, v, jnp.exp(jnp.minimum(v, 0.0)) - 1.0)


def _embed_body(z_ref, emb_ref, ws_ref, wt_ref, x_ref, xs_ref, xt_ref):
    z = z_ref[...]
    ids = lax.broadcasted_iota(jnp.int32, (1, emb_ref.shape[0]), 1)
    oh = (z == ids).astype(jnp.float32)
    x = jnp.dot(oh, emb_ref[...], preferred_element_type=jnp.float32)
    x_ref[...] = x
    xs_ref[...] = jnp.dot(x, ws_ref[...], preferred_element_type=jnp.float32)
    xt_ref[...] = jnp.dot(x, wt_ref[...], preferred_element_type=jnp.float32)


def _sc_gather_body(xs_hbm, xt_hbm, src_hbm, dst_hbm, gs_hbm, gt_hbm,
                    idx_s0, idx_t0, rows_s0, rows_t0, idx_s1, idx_t1,
                    rows_s1, rows_t1, sem_s0, sem_t0, sem_s1, sem_t1):
    wid = lax.axis_index("s") * 2 + lax.axis_index("c")
    nchunks = src_hbm.shape[0] // _CW
    iters = (nchunks + _NW - 1) // _NW
    bufs = ((idx_s0, idx_t0, rows_s0, rows_t0, sem_s0, sem_t0),
            (idx_s1, idx_t1, rows_s1, rows_t1, sem_s1, sem_t1))

    def prefetch(j, b):
        idx_s, idx_t, rows_s, rows_t, sem_s, sem_t = bufs[b]
        c = j * _NW + wid

        @pl.when(c < nchunks)
        def _():
            pltpu.sync_copy(src_hbm.at[pl.ds(c * _CW, _CW)], idx_s)
            pltpu.sync_copy(dst_hbm.at[pl.ds(c * _CW, _CW)], idx_t)
            pltpu.async_copy(xs_hbm.at[idx_s], rows_s, sem_s)
            pltpu.async_copy(xt_hbm.at[idx_t], rows_t, sem_t)

    def drain(j, b):
        idx_s, idx_t, rows_s, rows_t, sem_s, sem_t = bufs[b]
        c = j * _NW + wid

        @pl.when(c < nchunks)
        def _():
            pltpu.make_async_copy(xs_hbm.at[idx_s], rows_s, sem_s).wait()
            pltpu.make_async_copy(xt_hbm.at[idx_t], rows_t, sem_t).wait()
            pltpu.sync_copy(rows_s, gs_hbm.at[pl.ds(c * _CW, _CW)])
            pltpu.sync_copy(rows_t, gt_hbm.at[pl.ds(c * _CW, _CW)])

    prefetch(0, 0)

    def body(j2, carry):
        j = j2 * 2
        prefetch(j + 1, 1)
        drain(j, 0)
        prefetch(j + 2, 0)
        drain(j + 1, 1)
        return carry

    lax.fori_loop(0, (iters + 1) // 2, body, 0)


def _edge_body(gs_ref, gt_ref, d_ref, wd_ref, w1_ref, w2_ref,
               b1_ref, b2_ref, means_ref, betas_ref, out_ref):
    d = d_ref[...]
    dcl = jnp.minimum(d, _DCUT)
    cutoff = 0.5 * (jnp.cos(dcl * (jnp.pi / _DCUT)) + 1.0)
    rbf = cutoff * jnp.exp(-betas_ref[...] * (jnp.exp(-d) - means_ref[...]) ** 2)
    dist_emb = jnp.tanh(jnp.dot(rbf, wd_ref[...], preferred_element_type=jnp.float32))
    msg = dist_emb * (gs_ref[...] + gt_ref[...])
    h = _elu(jnp.dot(msg, w1_ref[...], preferred_element_type=jnp.float32) + b1_ref[...])
    h = _elu(jnp.dot(h, w2_ref[...], preferred_element_type=jnp.float32) + b2_ref[...])
    out_ref[...] = h


def _sc_scatter_body(h_hbm, dst_hbm, zeros_hbm, out0_hbm, out1_hbm,
                     idxr0, rows0, idxr1, rows1, semi0, semr0, semi1, semr1,
                     acc):
    cid = lax.axis_index("c")
    sid = lax.axis_index("s")
    npad = zeros_hbm.shape[0]
    stripe = npad // 16
    sl = pl.ds(sid * stripe, stripe)
    pltpu.sync_copy(zeros_hbm.at[sl], acc.at[sl])
    plsc.subcore_barrier()

    nchunks = dst_hbm.shape[0] // _CW
    half = nchunks // 2
    iters = (half + 15) // 16
    bufs = ((idxr0, rows0, semi0, semr0), (idxr1, rows1, semi1, semr1))

    def prefetch(j, b):
        idxr, rows, semi, semr = bufs[b]
        k = j * 16 + sid

        @pl.when(k < half)
        def _():
            c = cid * half + k
            pltpu.async_copy(h_hbm.at[pl.ds(c * _CW, _CW)], rows, semr)
            pltpu.async_copy(dst_hbm.at[pl.ds(c * _CW, _CW)], idxr, semi)

    def process(j, b):
        idxr, rows, semi, semr = bufs[b]
        k = j * 16 + sid

        @pl.when(k < half)
        def _():
            c = cid * half + k
            pltpu.make_async_copy(h_hbm.at[pl.ds(c * _CW, _CW)], rows, semr).wait()
            pltpu.make_async_copy(dst_hbm.at[pl.ds(c * _CW, _CW)], idxr, semi).wait()
            pltpu.sync_copy(rows, acc.at[idxr], add=True)

    prefetch(0, 0)

    def body(j2, carry):
        j = j2 * 2
        prefetch(j + 1, 1)
        process(j, 0)
        prefetch(j + 2, 0)
        process(j + 1, 1)
        return carry

    lax.fori_loop(0, (iters + 1) // 2, body, 0)
    plsc.subcore_barrier()

    @pl.when(cid == 0)
    def _():
        pltpu.sync_copy(acc.at[sl], out0_hbm.at[sl])

    @pl.when(cid == 1)
    def _():
        pltpu.sync_copy(acc.at[sl], out1_hbm.at[sl])


def _node_body(with_proj, x_ref, p0_ref, p1_ref, bc_ref, br_ref,
               wrc_ref, wca_ref, bcomb_ref, gnw_ref, gnb_ref, gnms_ref,
               *rest):
    if with_proj:
        ws_ref, wt_ref, xo_ref, xs_ref, xt_ref = rest
    else:
        (xo_ref,) = rest
    x = x_ref[...]
    n = x.shape[0]
    aggr = p0_ref[...][:n] + p1_ref[...][:n]
    pre = _elu(jnp.dot(x, wrc_ref[...], preferred_element_type=jnp.float32)
               + jnp.dot(aggr, wca_ref[...], preferred_element_type=jnp.float32)
               + bcomb_ref[...])
    seg_row = lax.broadcasted_iota(jnp.int32, (1, _NSEG), 1)
    seg_col = lax.broadcasted_iota(jnp.int32, (_NSEG, 1), 0)
    oh = (bc_ref[...] == seg_row).astype(jnp.float32)
    oht = (seg_col == br_ref[...]).astype(jnp.float32)
    cnt = jnp.maximum(jnp.sum(oht, axis=1, keepdims=True), 1.0)
    mean = jnp.dot(oht, pre, preferred_element_type=jnp.float32) / cnt
    mb = jnp.dot(oh, mean * gnms_ref[...], preferred_element_type=jnp.float32)
    xc = pre - mb
    var = jnp.dot(oht, xc * xc, preferred_element_type=jnp.float32) / cnt
    inv = lax.rsqrt(var + 1e-5)
    invb = jnp.dot(oh, inv, preferred_element_type=jnp.float32)
    xn = x + gnw_ref[...] * xc * invb + gnb_ref[...]
    xo_ref[...] = xn
    if with_proj:
        xs_ref[...] = jnp.dot(xn, ws_ref[...], preferred_element_type=jnp.float32)
        xt_ref[...] = jnp.dot(xn, wt_ref[...], preferred_element_type=jnp.float32)


def _sc_mesh():
    return plsc.VectorSubcoreMesh(core_axis_name="c", subcore_axis_name="s")


def _gather_call(n, e):
    return pl.kernel(
        _sc_gather_body,
        out_type=[jax.ShapeDtypeStruct((e, _H), jnp.float32)] * 2,
        mesh=_sc_mesh(),
        scratch_types=[
            pltpu.VMEM((_CW,), jnp.int32),
            pltpu.VMEM((_CW,), jnp.int32),
            pltpu.VMEM((_CW, _H), jnp.float32),
            pltpu.VMEM((_CW, _H), jnp.float32),
            pltpu.VMEM((_CW,), jnp.int32),
            pltpu.VMEM((_CW,), jnp.int32),
            pltpu.VMEM((_CW, _H), jnp.float32),
            pltpu.VMEM((_CW, _H), jnp.float32),
            pltpu.SemaphoreType.DMA,
            pltpu.SemaphoreType.DMA,
            pltpu.SemaphoreType.DMA,
            pltpu.SemaphoreType.DMA,
        ],
    )


def _scatter_call(npad, e):
    return pl.kernel(
        _sc_scatter_body,
        out_type=[jax.ShapeDtypeStruct((npad, _H), jnp.float32)] * 2,
        mesh=_sc_mesh(),
        scratch_types=[
            pltpu.VMEM((_CW,), jnp.int32),
            pltpu.VMEM((_CW, _H), jnp.float32),
            pltpu.VMEM((_CW,), jnp.int32),
            pltpu.VMEM((_CW, _H), jnp.float32),
            pltpu.SemaphoreType.DMA,
            pltpu.SemaphoreType.DMA,
            pltpu.SemaphoreType.DMA,
            pltpu.SemaphoreType.DMA,
            pltpu.VMEM_SHARED((npad, _H), jnp.float32),
        ],
    )


def _edge_call(e):
    grid = e // _EB
    wspec = pl.BlockSpec((_H, _H), lambda i: (0, 0))
    vspec = pl.BlockSpec((1, _H), lambda i: (0, 0))
    espec = pl.BlockSpec((_EB, _H), lambda i: (i, 0))
    return pl.pallas_call(
        _edge_body,
        grid=(grid,),
        in_specs=[espec, espec, pl.BlockSpec((_EB, 1), lambda i: (i, 0)),
                  wspec, wspec, wspec, vspec, vspec, vspec, vspec],
        out_specs=espec,
        out_shape=jax.ShapeDtypeStruct((e, _H), jnp.float32),
    )


def kernel(z, edge_index, edge_weight, batch, params):
    n = z.shape[0]
    e = edge_index.shape[1]
    npad = ((n + 127) // 128) * 128
    src = edge_index[0].astype(jnp.int32)
    dst = edge_index[1].astype(jnp.int32)
    z2 = z.astype(jnp.int32).reshape(n, 1)
    bcol = batch.astype(jnp.int32).reshape(n, 1)
    brow = batch.astype(jnp.int32).reshape(1, n)
    zeros_nh = jnp.zeros((npad, _H), jnp.float32)

    start = jnp.exp(jnp.asarray(-_DCUT, jnp.float32))
    means = jnp.linspace(start, 1.0, _H).reshape(1, _H).astype(jnp.float32)
    betas = jnp.full((1, _H), (2.0 / _H * (1.0 - start)) ** (-2), jnp.float32)

    def prep(p):
        return dict(
            wd=p["Wdist"].T,
            we_s=p["Wedge"][:, :_H].T,
            we_t=p["Wedge"][:, _H:].T,
            w1=p["W1"].T,
            w2=p["W2"].T,
            b1=p["b1"].reshape(1, _H),
            b2=p["b2"].reshape(1, _H),
            wrc=p["Wres"].T + p["Wcomb"][:, :_H].T,
            wca=p["Wcomb"][:, _H:].T,
            bcomb=p["bcomb"].reshape(1, _H),
            gnw=p["gn_w"].reshape(1, _H),
            gnb=p["gn_b"].reshape(1, _H),
            gnms=p["gn_ms"].reshape(1, _H),
        )

    lp = [prep(p) for p in params["layers"]]

    nh = jax.ShapeDtypeStruct((n, _H), jnp.float32)
    x, xs, xt = pl.pallas_call(
        _embed_body, out_shape=(nh, nh, nh),
    )(z2, params["emb"], lp[0]["we_s"], lp[0]["we_t"])

    gather = _gather_call(n, e)
    scatter = _scatter_call(npad, e)
    edge = _edge_call(e)

    for l in range(2):
        p = lp[l]
        gs, gt = gather(xs, xt, src, dst)
        h = edge(gs, gt, edge_weight, p["wd"], p["w1"], p["w2"],
                 p["b1"], p["b2"], means, betas)
        p0, p1 = scatter(h, dst, zeros_nh)
        if l == 0:
            body = functools.partial(_node_body, True)
            x, xs, xt = pl.pallas_call(body, out_shape=(nh, nh, nh))(
                x, p0, p1, bcol, brow, p["wrc"], p["wca"], p["bcomb"],
                p["gnw"], p["gnb"], p["gnms"], lp[1]["we_s"], lp[1]["we_t"])
        else:
            body = functools.partial(_node_body, False)
            x = pl.pallas_call(body, out_shape=nh)(
                x, p0, p1, bcol, brow, p["wrc"], p["wca"], p["bcomb"],
                p["gnw"], p["gnb"], p["gnms"])
    return x
